# Initial kernel scaffold; baseline (speedup 1.0000x reference)
import functools

import jax
import jax.numpy as jnp
from jax import lax
from jax.experimental import pallas as pl
from jax.experimental.pallas import tpu as pltpu
from jax.experimental.pallas import tpu_sc as plsc

_NUM_USERS = 50000
_NUM_ITEMS = 50000
_N = _NUM_USERS + _NUM_ITEMS
_EMB = 16
_E = 3200000
_LAYERS = 3

_NC = 2
_NS = 16
_NW = _NC * _NS
_G = 128
_GPS = 16
_SUP = _G * _GPS
_K = -(-_E // (_NW * _SUP))
_E_PAD = _NW * _K * _SUP
_ROWS_PT = _N // _NS


def _layer_body(src_h, dst_h, w_h, table_h, part_h,
                sidx, didx, wv, rows, stage, acc_sh, sem):
    c = lax.axis_index("c")
    s = lax.axis_index("s")
    wid = s * _NC + c

    zero16 = jnp.zeros((_EMB,), jnp.float32)

    @pl.loop(0, _ROWS_PT)
    def _zero(i):
        stage[i, :] = zero16

    pltpu.sync_copy(stage, acc_sh.at[pl.ds(s * _ROWS_PT, _ROWS_PT)])
    plsc.subcore_barrier()

    @pl.loop(0, _K)
    def _super(k):
        base = (wid * _K + k) * _GPS
        pltpu.sync_copy(src_h.at[pl.ds(base, _GPS)], sidx)
        pltpu.sync_copy(dst_h.at[pl.ds(base, _GPS)], didx)
        pltpu.sync_copy(w_h.at[pl.ds(base, _GPS)], wv)
        for g in range(_GPS):
            pltpu.async_copy(table_h.at[sidx.at[g]], rows, sem).wait()
            gfull = jnp.full((16,), g, jnp.int32)

            @pl.loop(0, _G)
            def _scale(e):
                wspl = plsc.load_gather(wv, [gfull, jnp.full((16,), e, jnp.int32)])
                rows[e, :] = rows[e, :] * wspl

            pltpu.sync_copy(rows, acc_sh.at[didx.at[g]], add=True)

    plsc.subcore_barrier()

    r0 = s * _ROWS_PT
    pltpu.sync_copy(acc_sh.at[pl.ds(r0, _ROWS_PT)], stage)
    pltpu.sync_copy(stage, part_h.at[c].at[pl.ds(r0, _ROWS_PT)])


@jax.jit
def _layer(src2, dst2, w2, table):
    mesh = plsc.VectorSubcoreMesh(core_axis_name="c", subcore_axis_name="s")
    return pl.kernel(
        _layer_body,
        out_type=jax.ShapeDtypeStruct((_NC, _N, _EMB), jnp.float32),
        mesh=mesh,
        scratch_types=[
            pltpu.VMEM((_GPS, _G), jnp.int32),
            pltpu.VMEM((_GPS, _G), jnp.int32),
            pltpu.VMEM((_GPS, _G), jnp.float32),
            pltpu.VMEM((_G, _EMB), jnp.float32),
            pltpu.VMEM((_ROWS_PT, _EMB), jnp.float32),
            pltpu.VMEM_SHARED((_N, _EMB), jnp.float32),
            pltpu.SemaphoreType.DMA,
        ],
        name="lightgcn_sc_layer",
    )(src2, dst2, w2, table)


def _combine_body(scale, p_ref, f_ref, t_out, f_out):
    t = p_ref[0] + p_ref[1]
    t_out[...] = t
    f_out[...] = (f_ref[...] + t) * scale


@functools.partial(jax.jit, static_argnums=(2,))
def _combine(parts, fsum, scale):
    pv = parts.reshape(_NC, _N * _EMB // 128, 128)
    fv = fsum.reshape(_N * _EMB // 128, 128)
    t, f = pl.pallas_call(
        functools.partial(_combine_body, scale),
        out_shape=(
            jax.ShapeDtypeStruct(fv.shape, jnp.float32),
            jax.ShapeDtypeStruct(fv.shape, jnp.float32),
        ),
        name="lightgcn_combine",
    )(pv, fv)
    return t.reshape(_N, _EMB), f.reshape(_N, _EMB)


def kernel(edge_index, edge_weight, user_weight, item_weight):
    src = jnp.pad(edge_index[0], (0, _E_PAD - _E)).reshape(_E_PAD // _G, _G)
    dst = jnp.pad(edge_index[1], (0, _E_PAD - _E)).reshape(_E_PAD // _G, _G)
    w = jnp.pad(edge_weight, (0, _E_PAD - _E)).reshape(_E_PAD // _G, _G)

    table = jnp.concatenate([user_weight, item_weight], axis=0)
    fsum = table
    for layer in range(_LAYERS):
        parts = _layer(src, dst, w, table)
        scale = 1.0 / (_LAYERS + 1) if layer == _LAYERS - 1 else 1.0
        table, fsum = _combine(parts, fsum, scale)
    return fsum[:_NUM_USERS], fsum[_NUM_USERS:]


# speedup vs baseline: 9.8660x; 9.8660x over previous
import jax
import jax.numpy as jnp
from jax import lax
from jax.experimental import pallas as pl
from jax.experimental.pallas import tpu as pltpu
from jax.experimental.pallas import tpu_sc as plsc

_NUM_USERS = 50000
_NUM_ITEMS = 50000
_N = _NUM_USERS + _NUM_ITEMS
_EMB = 16
_E = 3200000
_LAYERS = 3

_NS = 16
_G = 128
_GPS = 8
_SUP = _G * _GPS
_K = -(-_E // (_NS * _SUP))
_E_PAD = _NS * _K * _SUP
_ROWS_PT = 6272
_N_PAD = _NS * _ROWS_PT
_CH = 784
_NCH = _ROWS_PT // _CH


def _body(src_h, dst_h, w_h, table0_h, final_h, table_h,
          sidx, didx, wv, rows, a_buf, f_buf, acc_sh, sem):
    s = lax.axis_index("s")

    @pl.loop(0, _NCH)
    def _init(j):
        r = pl.multiple_of(s * _ROWS_PT + j * _CH, 8)
        pltpu.sync_copy(table0_h.at[pl.ds(r, _CH)], a_buf)
        pltpu.sync_copy(a_buf, table_h.at[pl.ds(r, _CH)])
        pltpu.sync_copy(a_buf, final_h.at[pl.ds(r, _CH)])

    zero16 = jnp.zeros((_EMB,), jnp.float32)

    for layer in range(_LAYERS):
        @pl.loop(0, _CH)
        def _z(i):
            a_buf[i, :] = zero16

        @pl.loop(0, _NCH)
        def _zc(j):
            r = pl.multiple_of(s * _ROWS_PT + j * _CH, 8)
            pltpu.sync_copy(a_buf, acc_sh.at[pl.ds(r, _CH)])

        plsc.subcore_barrier()

        @pl.loop(0, _K)
        def _super(k):
            base = pl.multiple_of((s * _K + k) * _GPS, 8)
            wbase = pl.multiple_of((s * _K + k) * _SUP, 8)
            pltpu.sync_copy(src_h.at[pl.ds(base, _GPS)], sidx)
            pltpu.sync_copy(dst_h.at[pl.ds(base, _GPS)], didx)
            pltpu.sync_copy(w_h.at[pl.ds(wbase, _SUP)], wv)
            for g in range(_GPS):
                pltpu.async_copy(table_h.at[sidx.at[g]], rows, sem).wait()

                @pl.loop(0, _G // 16)
                def _scale(t):
                    off = pl.multiple_of(g * _G + t * 16, 8)
                    w16 = wv[pl.ds(off, 16)]
                    for l in range(16):
                        wspl = jnp.take_along_axis(
                            w16, jnp.full((16,), l, jnp.int32), axis=0)
                        e = t * 16 + l
                        rows[e, :] = rows[e, :] * wspl

                pltpu.sync_copy(rows, acc_sh.at[didx.at[g]], add=True)

        plsc.subcore_barrier()

        last = layer == _LAYERS - 1
        inv = 1.0 / (_LAYERS + 1)

        @pl.loop(0, _NCH)
        def _fold(j):
            r = pl.multiple_of(s * _ROWS_PT + j * _CH, 8)
            pltpu.sync_copy(acc_sh.at[pl.ds(r, _CH)], a_buf)
            pltpu.sync_copy(final_h.at[pl.ds(r, _CH)], f_buf)

            @pl.loop(0, _CH)
            def _add(i):
                if last:
                    f_buf[i, :] = (f_buf[i, :] + a_buf[i, :]) * inv
                else:
                    f_buf[i, :] = f_buf[i, :] + a_buf[i, :]

            if not last:
                pltpu.sync_copy(a_buf, table_h.at[pl.ds(r, _CH)])
            pltpu.sync_copy(f_buf, final_h.at[pl.ds(r, _CH)])

        plsc.subcore_barrier()


@jax.jit
def _run(src2, dst2, w, table0):
    mesh = plsc.VectorSubcoreMesh(core_axis_name="c", subcore_axis_name="s",
                                  num_cores=1)
    final, _ = pl.kernel(
        _body,
        out_type=(
            jax.ShapeDtypeStruct((_N_PAD, _EMB), jnp.float32),
            jax.ShapeDtypeStruct((_N_PAD, _EMB), jnp.float32),
        ),
        mesh=mesh,
        scratch_types=[
            pltpu.VMEM((_GPS, _G), jnp.int32),
            pltpu.VMEM((_GPS, _G), jnp.int32),
            pltpu.VMEM((_SUP,), jnp.float32),
            pltpu.VMEM((_G, _EMB), jnp.float32),
            pltpu.VMEM((_CH, _EMB), jnp.float32),
            pltpu.VMEM((_CH, _EMB), jnp.float32),
            pltpu.VMEM_SHARED((_N_PAD, _EMB), jnp.float32),
            pltpu.SemaphoreType.DMA,
        ],
        compiler_params=pltpu.CompilerParams(use_tc_tiling_on_sc=False),
        name="lightgcn_sc",
    )(src2, dst2, w, table0)
    return final


def kernel(edge_index, edge_weight, user_weight, item_weight):
    src = jnp.pad(edge_index[0], (0, _E_PAD - _E)).reshape(_E_PAD // _G, _G)
    dst = jnp.pad(edge_index[1], (0, _E_PAD - _E)).reshape(_E_PAD // _G, _G)
    w = jnp.pad(edge_weight, (0, _E_PAD - _E))
    table0 = jnp.pad(jnp.concatenate([user_weight, item_weight], axis=0),
                     ((0, _N_PAD - _N), (0, 0)))
    final = _run(src, dst, w, table0)
    return final[:_NUM_USERS], final[_NUM_USERS:_N]
